# baseline (speedup 1.0000x reference)
import functools

import jax
import jax.numpy as jnp
from jax import lax
from jax.experimental import pallas as pl
from jax.experimental.pallas import tpu as pltpu
from jax.experimental.pallas import tpu_sc as plsc

B = 4096
S = 200
D = 64
L = 16
CHUNK = 40
CPR = S // CHUNK
NW = 32
BPW = B // NW
TASKS = BPW * CPR
DV = D // (2 * L)
NBUF = 4


def _rne16(x):
    fb = lax.bitcast_convert_type(x, jnp.uint32)
    return (fb + jnp.uint32(0x7FFF) + ((fb >> 16) & jnp.uint32(1))) >> 16


def _packT_body(lo_ref, hi_ref, o_ref):
    w = _rne16(lo_ref[...]) | (_rne16(hi_ref[...]) << 16)
    o_ref[...] = lax.bitcast_convert_type(w, jnp.int32)


def _pack(emb):
    v = emb.shape[0]
    x = 64 if v % 64 == 0 else 32
    et = emb.T.reshape(D, v // x, x)
    wt = pl.pallas_call(
        _packT_body,
        grid=(D // 2,),
        in_specs=[
            pl.BlockSpec((1, v // x, x), lambda i: (i, 0, 0)),
            pl.BlockSpec((1, v // x, x), lambda i: (i + D // 2, 0, 0)),
        ],
        out_specs=pl.BlockSpec((1, v // x, x), lambda i: (i, 0, 0)),
        out_shape=jax.ShapeDtypeStruct((D // 2, v // x, x), jnp.int32),
    )(et, et)
    return wt.reshape(D // 2, v).T


def _make_pool(num_tables):
    owidth = num_tables * D

    def body(*refs):
        toks = refs[:num_tables]
        embs = refs[num_tables:2 * num_tables]
        out = refs[2 * num_tables]
        idx_v = refs[2 * num_tables + 1]
        rbufs = refs[2 * num_tables + 2:2 * num_tables + 2 + NBUF]
        stage = refs[2 * num_tables + 2 + NBUF]
        sem = refs[2 * num_tables + 3 + NBUF]

        cid = lax.axis_index("c")
        sid = lax.axis_index("s")
        wid = sid * 2 + cid

        def zbody(i, carry):
            z = jnp.zeros((L,), jnp.float32)
            for j in range(owidth // L):
                stage[i, pl.ds(L * j, L)] = z
            return carry

        lax.fori_loop(0, BPW, zbody, 0)

        himask = jnp.full((L,), -65536, jnp.int32)

        for t in range(num_tables):
            tok = toks[t]
            emb = embs[t]
            pltpu.sync_copy(tok.at[pl.ds(wid * TASKS, TASKS)], idx_v)

            def fire(k, rbuf, emb=emb):
                pltpu.make_async_copy(emb.at[idx_v.at[k]], rbuf, sem).start()

            def drain(k, rbuf, emb=emb):
                pltpu.make_async_copy(emb.at[idx_v.at[k]], rbuf, sem).wait()

            def accum(k, rbuf, t=t):
                acc = [jnp.zeros((L,), jnp.float32) for _ in range(8)]
                for s in range(CHUNK):
                    bank = (s % 2) * 4
                    for j in range(2):
                        w = rbuf[s, pl.ds(L * j, L)]
                        ev = plsc.bitcast(lax.shift_left(w, 16), jnp.float32)
                        od = plsc.bitcast(lax.bitwise_and(w, himask), jnp.float32)
                        acc[bank + 2 * j] = acc[bank + 2 * j] + ev
                        acc[bank + 2 * j + 1] = acc[bank + 2 * j + 1] + od
                b_loc = k // CPR
                for j in range(2):
                    for eo in range(2):
                        plsc.addupdate(
                            stage.at[b_loc, pl.ds(t * D + 32 * eo + L * j, L)],
                            acc[2 * j + eo] + acc[4 + 2 * j + eo],
                        )

            for p in range(NBUF - 1):
                fire(p, rbufs[p])

            def lbody(kk, carry):
                for p in range(NBUF):
                    k = NBUF * kk + p

                    drain(k, rbufs[p])

                    @pl.when(k + NBUF - 1 < TASKS)
                    def _(k=k, p=p):
                        fire(k + NBUF - 1, rbufs[(p + NBUF - 1) % NBUF])

                    accum(k, rbufs[p])
                return carry

            lax.fori_loop(0, TASKS // NBUF, lbody, 0)

        pltpu.sync_copy(stage, out.at[pl.ds(wid * BPW, BPW)])

    return functools.partial(
        pl.kernel,
        out_type=jax.ShapeDtypeStruct((B, owidth), jnp.float32),
        mesh=plsc.VectorSubcoreMesh(core_axis_name="c", subcore_axis_name="s"),
        scratch_types=(
            [pltpu.VMEM((TASKS, CHUNK), jnp.int32)]
            + [pltpu.VMEM((CHUNK, D // 2), jnp.int32) for _ in range(NBUF)]
            + [pltpu.VMEM((BPW, owidth), jnp.float32), pltpu.SemaphoreType.DMA]
        ),
        compiler_params=pltpu.CompilerParams(
            use_tc_tiling_on_sc=False, needs_layout_passes=False
        ),
    )(body)


_pool1 = _make_pool(1)
_pool2 = _make_pool(2)


def _mlp_body(x1_ref, x23_ref, w1_ref, b1_ref, w2_ref, b2_ref, o_ref):
    h = lax.dot_general(
        x1_ref[...], w1_ref[pl.ds(0, D), :], (((1,), (0,)), ((), ())),
        preferred_element_type=jnp.float32, precision=lax.Precision.HIGHEST,
    )
    h = h + lax.dot_general(
        x23_ref[...], w1_ref[pl.ds(D, 2 * D), :], (((1,), (0,)), ((), ())),
        preferred_element_type=jnp.float32, precision=lax.Precision.HIGHEST,
    )
    h = jnp.maximum(h * (1.0 / S) + b1_ref[...], 0.0)
    o = lax.dot_general(
        h, w2_ref[...], (((1,), (0,)), ((), ())),
        preferred_element_type=jnp.float32, precision=lax.Precision.HIGHEST,
    )
    o_ref[...] = o + b2_ref[...]


def _mlp(x1, x23, W1, b1, W2, b2):
    return pl.pallas_call(
        _mlp_body,
        out_shape=jax.ShapeDtypeStruct((B, W2.shape[1]), jnp.float32),
    )(x1, x23, W1, b1.reshape(1, -1), W2, b2.reshape(1, -1))


def kernel(tokens_1gram, tokens_2gram, tokens_3gram, emb1, emb2, emb3, W1, b1, W2, b2):
    t1 = tokens_1gram.reshape(-1, CHUNK)
    t2 = tokens_2gram.reshape(-1, CHUNK)
    t3 = tokens_3gram.reshape(-1, CHUNK)
    p2 = _pack(emb2)
    p3 = _pack(emb3)
    emb1b, p2, p3 = lax.optimization_barrier((emb1, p2, p3))
    p1 = _pack(emb1b)
    pooled23 = _pool2(t2, t3, p2, p3)
    pooled1 = _pool1(t1, p1)
    return _mlp(pooled1, pooled23, W1, b1, W2, b2)


# speedup vs baseline: 2.1307x; 2.1307x over previous
import functools

import jax
import jax.numpy as jnp
from jax import lax
from jax.experimental import pallas as pl
from jax.experimental.pallas import tpu as pltpu
from jax.experimental.pallas import tpu_sc as plsc

B = 4096
S = 200
D = 64
L = 16
CHUNK = 40
CPR = S // CHUNK
NW = 32
BPW = B // NW
TASKS = BPW * CPR
DV = D // L
NBUF = 5
NSEG = 2
TSEG = TASKS // NSEG


def _widen(emb):
    return jnp.pad(emb, ((0, 0), (0, D)))


def _make_pool(num_tables):
    owidth = num_tables * D

    def body(*refs):
        toks = refs[:num_tables]
        embs = refs[num_tables:2 * num_tables]
        out = refs[2 * num_tables]
        idx_v = refs[2 * num_tables + 1]
        rbufs = refs[2 * num_tables + 2:2 * num_tables + 2 + NBUF]
        stage = refs[2 * num_tables + 2 + NBUF]
        sem = refs[2 * num_tables + 3 + NBUF]

        cid = lax.axis_index("c")
        sid = lax.axis_index("s")
        wid = sid * 2 + cid

        def zbody(i, carry):
            z = jnp.zeros((L,), jnp.float32)
            for j in range(owidth // L):
                stage[i, pl.ds(L * j, L)] = z
            return carry

        lax.fori_loop(0, BPW, zbody, 0)

        for t in range(num_tables):
            tok = toks[t]
            emb = embs[t]

            def fire(k, rbuf, emb=emb):
                pltpu.make_async_copy(emb.at[idx_v.at[k]], rbuf, sem).start()

            def drain(k, rbuf, emb=emb):
                pltpu.make_async_copy(emb.at[idx_v.at[k]], rbuf, sem).wait()

            def accum(kg, rbuf, t=t):
                acc = [jnp.zeros((L,), jnp.float32) for _ in range(2 * DV)]
                for s in range(CHUNK):
                    bank = (s % 2) * DV
                    for j in range(DV):
                        acc[bank + j] = acc[bank + j] + rbuf[s, pl.ds(L * j, L)]
                b_loc = kg // CPR
                for j in range(DV):
                    plsc.addupdate(
                        stage.at[b_loc, pl.ds(t * D + L * j, L)],
                        acc[j] + acc[DV + j],
                    )

            def seg_body(h, carry):
                pltpu.sync_copy(
                    tok.at[pl.ds(wid * TASKS + h * TSEG, TSEG)], idx_v
                )
                for p in range(NBUF - 1):
                    fire(p, rbufs[p])

                def lbody(kk, carry2):
                    for p in range(NBUF):
                        k = NBUF * kk + p

                        drain(k, rbufs[p])

                        @pl.when(k + NBUF - 1 < TSEG)
                        def _(k=k, p=p):
                            fire(k + NBUF - 1, rbufs[(p + NBUF - 1) % NBUF])

                        accum(h * TSEG + k, rbufs[p])
                    return carry2

                lax.fori_loop(0, TSEG // NBUF, lbody, 0)
                return carry

            lax.fori_loop(0, NSEG, seg_body, 0)

        pltpu.sync_copy(stage, out.at[pl.ds(wid * BPW, BPW)])

    return functools.partial(
        pl.kernel,
        out_type=jax.ShapeDtypeStruct((B, owidth), jnp.float32),
        mesh=plsc.VectorSubcoreMesh(core_axis_name="c", subcore_axis_name="s"),
        scratch_types=(
            [pltpu.VMEM((TSEG, CHUNK), jnp.int32)]
            + [pltpu.VMEM((CHUNK, 2 * D), jnp.float32) for _ in range(NBUF)]
            + [pltpu.VMEM((BPW, owidth), jnp.float32), pltpu.SemaphoreType.DMA]
        ),
    )(body)


_pool1 = _make_pool(1)
_pool2 = _make_pool(2)


def _mlp_body(x1_ref, x23_ref, w1_ref, b1_ref, w2_ref, b2_ref, o_ref):
    h = lax.dot_general(
        x1_ref[...], w1_ref[pl.ds(0, D), :], (((1,), (0,)), ((), ())),
        preferred_element_type=jnp.float32, precision=lax.Precision.HIGHEST,
    )
    h = h + lax.dot_general(
        x23_ref[...], w1_ref[pl.ds(D, 2 * D), :], (((1,), (0,)), ((), ())),
        preferred_element_type=jnp.float32, precision=lax.Precision.HIGHEST,
    )
    h = jnp.maximum(h * (1.0 / S) + b1_ref[...], 0.0)
    o = lax.dot_general(
        h, w2_ref[...], (((1,), (0,)), ((), ())),
        preferred_element_type=jnp.float32, precision=lax.Precision.HIGHEST,
    )
    o_ref[...] = o + b2_ref[...]


def _mlp(x1, x23, W1, b1, W2, b2):
    return pl.pallas_call(
        _mlp_body,
        out_shape=jax.ShapeDtypeStruct((B, W2.shape[1]), jnp.float32),
    )(x1, x23, W1, b1.reshape(1, -1), W2, b2.reshape(1, -1))


def kernel(tokens_1gram, tokens_2gram, tokens_3gram, emb1, emb2, emb3, W1, b1, W2, b2):
    t1 = tokens_1gram.reshape(-1, CHUNK)
    t2 = tokens_2gram.reshape(-1, CHUNK)
    t3 = tokens_3gram.reshape(-1, CHUNK)
    p2 = _widen(emb2)
    p3 = _widen(emb3)
    p1 = _widen(emb1)
    pooled23 = _pool2(t2, t3, p2, p3)
    pooled1 = _pool1(t1, p1)
    return _mlp(pooled1, pooled23, W1, b1, W2, b2)


# speedup vs baseline: 2.3095x; 1.0839x over previous
import functools

import jax
import jax.numpy as jnp
from jax import lax
from jax.experimental import pallas as pl
from jax.experimental.pallas import tpu as pltpu
from jax.experimental.pallas import tpu_sc as plsc

B = 4096
S = 200
D = 64
L = 16
CHUNK = 40
CPR = S // CHUNK
NW = 32
BPW = B // NW
TASKS = BPW * CPR
DV = D // L
NBUF = 5
NSEG = 2
TSEG = TASKS // NSEG


def _widen(emb):
    return jnp.pad(emb, ((0, 0), (0, D)))


def _pack(emb):
    bits = lax.bitcast_convert_type(emb.astype(jnp.bfloat16), jnp.uint16)
    lo = bits[:, :32].astype(jnp.uint32)
    hi = bits[:, 32:].astype(jnp.uint32)
    return lax.bitcast_convert_type(lo | (hi << 16), jnp.int32)


def _make_pool(num_tables, packed):
    owidth = num_tables * D

    def body(*refs):
        toks = refs[:num_tables]
        embs = refs[num_tables:2 * num_tables]
        out = refs[2 * num_tables]
        idx_v = refs[2 * num_tables + 1]
        rbufs = refs[2 * num_tables + 2:2 * num_tables + 2 + NBUF]
        stage = refs[2 * num_tables + 2 + NBUF]
        sem = refs[2 * num_tables + 3 + NBUF]

        cid = lax.axis_index("c")
        sid = lax.axis_index("s")
        wid = sid * 2 + cid

        def zbody(i, carry):
            z = jnp.zeros((L,), jnp.float32)
            for j in range(owidth // L):
                stage[i, pl.ds(L * j, L)] = z
            return carry

        lax.fori_loop(0, BPW, zbody, 0)

        himask = jnp.full((L,), -65536, jnp.int32)

        for t in range(num_tables):
            tok = toks[t]
            emb = embs[t]

            def fire(k, rbuf, emb=emb):
                pltpu.make_async_copy(emb.at[idx_v.at[k]], rbuf, sem).start()

            def drain(k, rbuf, emb=emb):
                pltpu.make_async_copy(emb.at[idx_v.at[k]], rbuf, sem).wait()

            if packed:
                def accum(kg, rbuf, t=t):
                    acc = [jnp.zeros((L,), jnp.float32) for _ in range(8)]
                    for s in range(CHUNK):
                        bank = (s % 2) * 4
                        for j in range(2):
                            w = rbuf[s, pl.ds(L * j, L)]
                            ev = plsc.bitcast(lax.shift_left(w, 16), jnp.float32)
                            od = plsc.bitcast(
                                lax.bitwise_and(w, himask), jnp.float32
                            )
                            acc[bank + 2 * j] = acc[bank + 2 * j] + ev
                            acc[bank + 2 * j + 1] = acc[bank + 2 * j + 1] + od
                    b_loc = kg // CPR
                    for j in range(2):
                        for eo in range(2):
                            plsc.addupdate(
                                stage.at[
                                    b_loc, pl.ds(t * D + 32 * eo + L * j, L)
                                ],
                                acc[2 * j + eo] + acc[4 + 2 * j + eo],
                            )
            else:
                def accum(kg, rbuf, t=t):
                    acc = [jnp.zeros((L,), jnp.float32) for _ in range(2 * DV)]
                    for s in range(CHUNK):
                        bank = (s % 2) * DV
                        for j in range(DV):
                            acc[bank + j] = (
                                acc[bank + j] + rbuf[s, pl.ds(L * j, L)]
                            )
                    b_loc = kg // CPR
                    for j in range(DV):
                        plsc.addupdate(
                            stage.at[b_loc, pl.ds(t * D + L * j, L)],
                            acc[j] + acc[DV + j],
                        )

            def seg_body(h, carry):
                pltpu.sync_copy(
                    tok.at[pl.ds(wid * TASKS + h * TSEG, TSEG)], idx_v
                )
                for p in range(NBUF - 1):
                    fire(p, rbufs[p])

                def lbody(kk, carry2):
                    for p in range(NBUF):
                        k = NBUF * kk + p

                        drain(k, rbufs[p])

                        @pl.when(k + NBUF - 1 < TSEG)
                        def _(k=k, p=p):
                            fire(k + NBUF - 1, rbufs[(p + NBUF - 1) % NBUF])

                        accum(h * TSEG + k, rbufs[p])
                    return carry2

                lax.fori_loop(0, TSEG // NBUF, lbody, 0)
                return carry

            lax.fori_loop(0, NSEG, seg_body, 0)

        pltpu.sync_copy(stage, out.at[pl.ds(wid * BPW, BPW)])

    if packed:
        rbuf_t = pltpu.VMEM((CHUNK, D // 2), jnp.int32)
        params = pltpu.CompilerParams(
            use_tc_tiling_on_sc=False, needs_layout_passes=False
        )
    else:
        rbuf_t = pltpu.VMEM((CHUNK, 2 * D), jnp.float32)
        params = None
    return functools.partial(
        pl.kernel,
        out_type=jax.ShapeDtypeStruct((B, owidth), jnp.float32),
        mesh=plsc.VectorSubcoreMesh(core_axis_name="c", subcore_axis_name="s"),
        scratch_types=(
            [pltpu.VMEM((TSEG, CHUNK), jnp.int32)]
            + [rbuf_t for _ in range(NBUF)]
            + [pltpu.VMEM((BPW, owidth), jnp.float32), pltpu.SemaphoreType.DMA]
        ),
        compiler_params=params,
    )(body)


_pool1 = _make_pool(1, packed=False)
_pool2 = _make_pool(2, packed=True)


def _mlp_body(x1_ref, x23_ref, w1_ref, b1_ref, w2_ref, b2_ref, o_ref):
    h = lax.dot_general(
        x1_ref[...], w1_ref[pl.ds(0, D), :], (((1,), (0,)), ((), ())),
        preferred_element_type=jnp.float32, precision=lax.Precision.HIGHEST,
    )
    h = h + lax.dot_general(
        x23_ref[...], w1_ref[pl.ds(D, 2 * D), :], (((1,), (0,)), ((), ())),
        preferred_element_type=jnp.float32, precision=lax.Precision.HIGHEST,
    )
    h = jnp.maximum(h * (1.0 / S) + b1_ref[...], 0.0)
    o = lax.dot_general(
        h, w2_ref[...], (((1,), (0,)), ((), ())),
        preferred_element_type=jnp.float32, precision=lax.Precision.HIGHEST,
    )
    o_ref[...] = o + b2_ref[...]


def _mlp(x1, x23, W1, b1, W2, b2):
    return pl.pallas_call(
        _mlp_body,
        out_shape=jax.ShapeDtypeStruct((B, W2.shape[1]), jnp.float32),
    )(x1, x23, W1, b1.reshape(1, -1), W2, b2.reshape(1, -1))


def kernel(tokens_1gram, tokens_2gram, tokens_3gram, emb1, emb2, emb3, W1, b1, W2, b2):
    t1 = tokens_1gram.reshape(-1, CHUNK)
    t2 = tokens_2gram.reshape(-1, CHUNK)
    t3 = tokens_3gram.reshape(-1, CHUNK)
    p2 = _pack(emb2)
    p3 = _pack(emb3)
    p1 = _widen(emb1)
    pooled23 = _pool2(t2, t3, p2, p3)
    pooled1 = _pool1(t1, p1)
    return _mlp(pooled1, pooled23, W1, b1, W2, b2)
